# baseline (speedup 1.0000x reference)
import math

import jax
import jax.numpy as jnp
from jax.experimental import pallas as pl
from jax.experimental.pallas import tpu as pltpu

N, K, DIN, H = 16, 4096, 1024, 64
KH = K // 2


def _gelu(z):
    return z * 0.5 * (1.0 + jax.lax.erf(z * (1.0 / math.sqrt(2.0))))


def _fused_kernel(x1_ref, x2_ref, maskf_ref, qh_ref, Wp_ref, bp_ref,
                  Wq_ref, bq_ref, Wk_ref, bk_ref, Wv_ref, bv_ref,
                  wsi_ref, attn_ref):
    Wp, bp = Wp_ref[...], bp_ref[...]
    Wk, bk = Wk_ref[...], bk_ref[...]
    Wv, bv = Wv_ref[...], bv_ref[...]
    q = qh_ref[0] @ Wq_ref[...] + bq_ref[...]

    p1 = _gelu(x1_ref[0] @ Wp + bp)
    k1 = p1 @ Wk + bk
    v1 = p1 @ Wv + bv
    s1 = jax.lax.dot_general(q, k1, (((1,), (1,)), ((), ())))
    p2 = _gelu(x2_ref[0] @ Wp + bp)
    k2 = p2 @ Wk + bk
    v2 = p2 @ Wv + bv
    s2 = jax.lax.dot_general(q, k2, (((1,), (1,)), ((), ())))

    scale = 1.0 / math.sqrt(H)
    maskf = maskf_ref[0]
    s1 = jnp.where(maskf[:, :KH] > 0, s1 * scale, -jnp.inf)
    s2 = jnp.where(maskf[:, KH:] > 0, s2 * scale, -jnp.inf)
    m = jnp.maximum(jnp.max(s1, axis=1, keepdims=True),
                    jnp.max(s2, axis=1, keepdims=True))
    e1 = jnp.exp(s1 - m)
    e2 = jnp.exp(s2 - m)
    l = (jnp.sum(e1, axis=1, keepdims=True)
         + jnp.sum(e2, axis=1, keepdims=True))
    a1 = e1 / l
    a2 = e2 / l
    attn_ref[0, :, :KH] = a1
    attn_ref[0, :, KH:] = a2
    wsi_ref[0] = a1 @ v1 + a2 @ v2


@jax.jit
def kernel(patches, mask, query_h, W_patch, b_patch, Wq, bq, Wk, bk, Wv, bv):
    maskf = mask.astype(jnp.float32).reshape(N, 1, K)
    full = lambda shape: pl.BlockSpec(shape, lambda n: (0,) * len(shape))
    wsi, attn = pl.pallas_call(
        _fused_kernel,
        grid=(N,),
        in_specs=[
            pl.BlockSpec((1, KH, DIN), lambda n: (n, 0, 0)),
            pl.BlockSpec((1, KH, DIN), lambda n: (n, 1, 0)),
            pl.BlockSpec((1, 1, K), lambda n: (n, 0, 0)),
            pl.BlockSpec((1, 1, H), lambda n: (n, 0, 0)),
            full((DIN, H)),
            full((1, H)),
            full((H, H)), full((1, H)),
            full((H, H)), full((1, H)),
            full((H, H)), full((1, H)),
        ],
        out_specs=[
            pl.BlockSpec((1, 1, H), lambda n: (n, 0, 0)),
            pl.BlockSpec((1, 1, K), lambda n: (n, 0, 0)),
        ],
        out_shape=[
            jax.ShapeDtypeStruct((N, 1, H), jnp.float32),
            jax.ShapeDtypeStruct((N, 1, K), jnp.float32),
        ],
        compiler_params=pltpu.CompilerParams(
            dimension_semantics=("arbitrary",),
        ),
    )(patches, patches, maskf, query_h.reshape(N, 1, H),
      W_patch, b_patch.reshape(1, H),
      Wq, bq.reshape(1, H), Wk, bk.reshape(1, H), Wv, bv.reshape(1, H))
    return (wsi.reshape(N, H), attn.reshape(N, K))
